# Initial kernel scaffold; baseline (speedup 1.0000x reference)
import jax
import jax.numpy as jnp
from jax.experimental import pallas as pl


def kernel(source, target, params):
    raise NotImplementedError("write your pallas kernel here")


# speedup vs baseline: 1.2176x; 1.2176x over previous
import functools
import math

import jax
import jax.numpy as jnp
import numpy as np
from jax import lax
from jax.experimental import pallas as pl

_B, _S, _D, _H, _DEPTH, _V = 4, 512, 512, 8, 6, 256
_DH = _D // _H
_NB = 16
_CH = 64
_NC = _S // _CH
_FF = 4 * _D
_M = _B * _S

_F32 = jnp.float32
_PREC = jax.lax.Precision.HIGHEST
_BF16 = jnp.bfloat16


def _dot_bf(a, b, dims=None):
    a = a.astype(_BF16)
    b = b.astype(_BF16)
    if dims is None:
        dims = (((a.ndim - 1,), (0,)), ((), ()))
    return lax.dot_general(a, b, dims, preferred_element_type=_F32)


def _sinusoid_np():
    pos = np.arange(_S)[:, None].astype(np.float64)
    i = np.arange(_D)[None, :].astype(np.float64)
    angle = pos / np.power(10000.0, (2.0 * (i // 2)) / _D)
    pe = np.zeros((_S, _D))
    pe[:, 0::2] = np.sin(angle[:, 0::2])
    pe[:, 1::2] = np.cos(angle[:, 1::2])
    return pe.astype(np.float32)


_PE = jnp.asarray(_sinusoid_np())


def _pc(*a, **k):
    return pl.pallas_call(*a, **k)


def _iota(shape, dim):
    return lax.broadcasted_iota(jnp.int32, shape, dim).astype(_F32)


def _ln_rows(xb):
    m = jnp.mean(xb, axis=1, keepdims=True)
    xc = xb - m
    v = jnp.mean(xc * xc, axis=1, keepdims=True)
    return xc / jnp.sqrt(v + 1e-5)


def _lnmm_body(act, x_ref, m_ref, s_ref, w_ref, o_ref):
    y = (x_ref[...] - m_ref[...]) / s_ref[...]
    out = _dot_bf(y, w_ref[...])
    if act == "gelu":
        out = jax.nn.gelu(out)
    o_ref[...] = out


def _lnmm(x, w, act=None, rtile=256, ntile=1024):
    m, k = x.shape
    n = w.shape[1]
    nt = min(n, ntile)
    x3 = x.reshape(_B, _S, k)
    mu = x3.mean(-1, keepdims=True).reshape(m, 1)
    sd = jnp.sqrt(x3.var(-1, keepdims=True) + 1e-5).reshape(m, 1)
    return _pc(
        functools.partial(_lnmm_body, act),
        grid=(m // rtile, n // nt),
        in_specs=[
            pl.BlockSpec((rtile, k), lambda i, j: (i, 0)),
            pl.BlockSpec((rtile, 1), lambda i, j: (i, 0)),
            pl.BlockSpec((rtile, 1), lambda i, j: (i, 0)),
            pl.BlockSpec((k, nt), lambda i, j: (0, j)),
        ],
        out_specs=pl.BlockSpec((rtile, nt), lambda i, j: (i, j)),
        out_shape=jax.ShapeDtypeStruct((m, n), _F32),
    )(x, mu, sd, w)


def _mm_body(a_ref, w_ref, o_ref):
    o_ref[...] = _dot_bf(a_ref[...], w_ref[...])


def _mm(a, w, rtile=256):
    m, k = a.shape
    n = w.shape[1]
    return _pc(
        _mm_body,
        grid=(m // rtile,),
        in_specs=[
            pl.BlockSpec((rtile, k), lambda i: (i, 0)),
            pl.BlockSpec((k, n), lambda i: (0, 0)),
        ],
        out_specs=pl.BlockSpec((rtile, n), lambda i: (i, 0)),
        out_shape=jax.ShapeDtypeStruct((m, n), _F32),
    )(a, w)


def _lsh_head_stage1(qk, v, nrm_h, rot, causal):
    rotated = _dot_bf(qk, rot)
    scores = jnp.concatenate([rotated, -rotated], axis=1)
    mx = jnp.max(scores, axis=1, keepdims=True)
    colio = _iota((_S, _NB), 1)
    cand = jnp.where(scores >= mx, colio, float(_NB))
    bidx = jnp.min(cand, axis=1, keepdims=True)
    bh = (colio == bidx).astype(_F32)

    counts = jnp.sum(bh, axis=0, keepdims=True)
    a16 = _iota((_NB, _NB), 0)
    b16 = _iota((_NB, _NB), 1)
    mlt = (b16 < a16).astype(_F32)
    below_mask = jnp.dot(bh, mlt, preferred_element_type=_F32, precision=_PREC)
    below = jnp.sum(below_mask * counts, axis=1, keepdims=True)

    i0 = _iota((_S, _S), 0)
    i1 = _iota((_S, _S), 1)
    tri = (i1 < i0).astype(_F32)
    cum_bh = jnp.dot(tri, bh, preferred_element_type=_F32, precision=_PREC)
    prefix = jnp.sum(bh * cum_bh, axis=1, keepdims=True)
    rank = below + prefix

    pt = (i1 == rank).astype(_F32)

    tA = (((0,), (0,)), ((), ()))
    sqk = lax.dot_general(pt, qk, tA, preferred_element_type=_F32, precision=_PREC)
    sv = lax.dot_general(pt, v, tA, preferred_element_type=_F32, precision=_PREC)
    spos = lax.dot_general(pt, _iota((_S, 1), 0), tA,
                           preferred_element_type=_F32, precision=_PREC)
    spos_t = jnp.dot(_iota((1, _S), 1), pt,
                     preferred_element_type=_F32, precision=_PREC)

    snorm = lax.dot_general(pt, nrm_h, tA,
                            preferred_element_type=_F32, precision=_PREC)
    sk = sqk / (snorm + 1e-9)

    scale = 1.0 / math.sqrt(_DH)
    tB = (((1,), (1,)), ((), ()))
    dots_chunks = []
    for c in range(_NC):
        pchunk = (c - 1) % _NC
        sl_c = slice(c * _CH, (c + 1) * _CH)
        sl_p = slice(pchunk * _CH, (pchunk + 1) * _CH)
        q_c = sqk[sl_c, :]
        k2 = jnp.concatenate([sk[sl_p, :], sk[sl_c, :]], axis=0)
        pq = spos[sl_c, :]
        pk = jnp.concatenate([spos_t[:, sl_p], spos_t[:, sl_c]], axis=1)
        dots = _dot_bf(q_c, k2, tB) * scale
        dots = jnp.where(pq == pk, -1e5, dots)
        if causal:
            dots = jnp.where(pq < pk, -1e9, dots)
        dots_chunks.append(dots)
    dots_all = jnp.concatenate(dots_chunks, axis=0)
    return dots_all, sv, rank


def _lsh1_body(causal, qkv_ref, nrm_ref, rot_ref, d_ref, sv_ref, rk_ref):
    qkv = qkv_ref[0]
    nrm = nrm_ref[0]
    ds, svs, rks = [], [], []
    for h in range(_H):
        qk = qkv[:, h * _DH:(h + 1) * _DH]
        v = qkv[:, _D + h * _DH:_D + (h + 1) * _DH]
        d, sv, rk = _lsh_head_stage1(qk, v, nrm[:, h:h + 1], rot_ref[h],
                                     causal)
        ds.append(d)
        svs.append(sv)
        rks.append(rk)
    d_ref[0] = jnp.concatenate(ds, axis=1)
    sv_ref[0] = jnp.concatenate(svs, axis=1)
    rk_ref[0] = jnp.concatenate(rks, axis=1)


def _lsh2_body(attn_ref, sv_ref, rk_ref, o_ref):
    attn = attn_ref[0]
    svb = sv_ref[0]
    rkb = rk_ref[0]
    i1 = _iota((_S, _S), 1)
    heads = []
    for h in range(_H):
        attn_h = attn[:, h * 2 * _CH:(h + 1) * 2 * _CH]
        sv_h = svb[:, h * _DH:(h + 1) * _DH]
        pt = (i1 == rkb[:, h:h + 1]).astype(_F32)
        outs = []
        for c in range(_NC):
            pchunk = (c - 1) % _NC
            sl_c = slice(c * _CH, (c + 1) * _CH)
            sl_p = slice(pchunk * _CH, (pchunk + 1) * _CH)
            v2 = jnp.concatenate([sv_h[sl_p, :], sv_h[sl_c, :]], axis=0)
            outs.append(_dot_bf(attn_h[sl_c, :], v2))
        so = jnp.concatenate(outs, axis=0)
        heads.append(jnp.dot(pt, so, preferred_element_type=_F32,
                             precision=_PREC))
    o_ref[0] = jnp.concatenate(heads, axis=1)


def _lsh(qkv3, rot, causal):
    nrm3 = jnp.linalg.norm(
        qkv3[:, :, :_D].reshape(_B, _S, _H, _DH), axis=-1)
    dots, sv, rk = _pc(
        functools.partial(_lsh1_body, causal),
        grid=(_B,),
        in_specs=[
            pl.BlockSpec((1, _S, 2 * _D), lambda b: (b, 0, 0)),
            pl.BlockSpec((1, _S, _H), lambda b: (b, 0, 0)),
            pl.BlockSpec((_H, _DH, _NB // 2), lambda b: (0, 0, 0)),
        ],
        out_specs=[
            pl.BlockSpec((1, _S, _H * 2 * _CH), lambda b: (b, 0, 0)),
            pl.BlockSpec((1, _S, _D), lambda b: (b, 0, 0)),
            pl.BlockSpec((1, _S, _H), lambda b: (b, 0, 0)),
        ],
        out_shape=[
            jax.ShapeDtypeStruct((_B, _S, _H * 2 * _CH), _F32),
            jax.ShapeDtypeStruct((_B, _S, _D), _F32),
            jax.ShapeDtypeStruct((_B, _S, _H), _F32),
        ],
    )(qkv3, nrm3, rot)
    attn = jax.nn.softmax(
        dots.reshape(_B, _S, _H, 2 * _CH), axis=-1).reshape(
            _B, _S, _H * 2 * _CH)
    return _pc(
        _lsh2_body,
        grid=(_B,),
        in_specs=[
            pl.BlockSpec((1, _S, _H * 2 * _CH), lambda b: (b, 0, 0)),
            pl.BlockSpec((1, _S, _D), lambda b: (b, 0, 0)),
            pl.BlockSpec((1, _S, _H), lambda b: (b, 0, 0)),
        ],
        out_specs=pl.BlockSpec((1, _S, _D), lambda b: (b, 0, 0)),
        out_shape=jax.ShapeDtypeStruct((_B, _S, _D), _F32),
    )(attn, sv, rk)


def _cross1_body(q_ref, k_ref, o_ref):
    scale = 1.0 / math.sqrt(_DH)
    tB = (((1,), (1,)), ((), ()))
    heads = []
    for h in range(_H):
        sl = slice(h * _DH, (h + 1) * _DH)
        heads.append(_dot_bf(q_ref[0][:, sl], k_ref[0][:, sl], tB) * scale)
    o_ref[0] = jnp.concatenate(heads, axis=1)


def _cross2_body(attn_ref, v_ref, o_ref):
    heads = []
    for h in range(_H):
        attn = attn_ref[0][:, h * _S:(h + 1) * _S]
        v = v_ref[0][:, h * _DH:(h + 1) * _DH]
        heads.append(_dot_bf(attn, v))
    o_ref[0] = jnp.concatenate(heads, axis=1)


def _cross(q3, kv3, layer):
    dots = _pc(
        _cross1_body,
        grid=(_B,),
        in_specs=[
            pl.BlockSpec((1, _S, _D), lambda b: (b, 0, 0)),
            pl.BlockSpec((1, _S, _D), lambda b: (b, 0, layer)),
        ],
        out_specs=pl.BlockSpec((1, _S, _H * _S), lambda b: (b, 0, 0)),
        out_shape=jax.ShapeDtypeStruct((_B, _S, _H * _S), _F32),
    )(q3, kv3)
    attn = jax.nn.softmax(
        dots.reshape(_B, _S, _H, _S), axis=-1).reshape(_B, _S, _H * _S)
    return _pc(
        _cross2_body,
        grid=(_B,),
        in_specs=[
            pl.BlockSpec((1, _S, _H * _S), lambda b: (b, 0, 0)),
            pl.BlockSpec((1, _S, _D), lambda b: (b, 0, _DEPTH + layer)),
        ],
        out_specs=pl.BlockSpec((1, _S, _D), lambda b: (b, 0, 0)),
        out_shape=jax.ShapeDtypeStruct((_B, _S, _D), _F32),
    )(attn, kv3)


def kernel(source, target, params):
    p = params


    x = (p['emb_enc'][source] + _PE[None]).reshape(_M, _D)
    for l in range(_DEPTH):
        wqkv = jnp.concatenate([p['enc_Wqk'][l], p['enc_Wv'][l]], axis=1)
        qkv = _lnmm(x, wqkv)
        a = _lsh(qkv.reshape(_B, _S, 2 * _D), p['enc_rot'][l], causal=False)
        x = x + _mm(a.reshape(_M, _D), p['enc_Wo'][l])
        h1 = _lnmm(x, p['enc_W1'][l], act="gelu")
        x = x + _mm(h1, p['enc_W2'][l])
    enc_x = x

    wkv_all = jnp.concatenate(
        [p['dec_Wck'][l] for l in range(_DEPTH)]
        + [p['dec_Wcv'][l] for l in range(_DEPTH)], axis=1)
    kv_all = _lnmm(enc_x, wkv_all)
    kv3 = kv_all.reshape(_B, _S, 2 * _DEPTH * _D)

    x = (p['emb_dec'][target] + _PE[None]).reshape(_M, _D)
    for l in range(_DEPTH):
        wqkv = jnp.concatenate([p['dec_Wqk'][l], p['dec_Wv'][l]], axis=1)
        qkv = _lnmm(x, wqkv)
        a = _lsh(qkv.reshape(_B, _S, 2 * _D), p['dec_rot'][l], causal=True)
        x = x + _mm(a.reshape(_M, _D), p['dec_Wo'][l])

        q = _lnmm(x, p['dec_Wcq'][l])
        c = _cross(q.reshape(_B, _S, _D), kv3, l)
        x = x + _mm(c.reshape(_M, _D), p['dec_Wco'][l])

        h1 = _lnmm(x, p['dec_W1'][l], act="gelu")
        x = x + _mm(h1, p['dec_W2'][l])

    logits = _lnmm(x, p['Wout'], ntile=256)
    return logits.reshape(_B, _S, _V)
